# baseline (speedup 1.0000x reference)
import math

import jax
import jax.numpy as jnp
from jax.experimental import pallas as pl
from jax.experimental.pallas import tpu as pltpu

D = 1024
ROWS = 2 * 4096 - 1
N = ROWS * D
BATCH = 4
CHUNK = 256 * D
NBLK = (N + CHUNK - 1) // CHUNK
NBUF = 4
LOOKAHEAD = 2
SCALE = math.sqrt(D)


def _base(k):
    return min(k * CHUNK, N - CHUNK)


def _body(w_hbm, o_hbm, bufs, sin_ref, s0, s1, s2, s3):
    sout = [s0, s1, s2, s3]

    def issue_gather(k):
        return pltpu.make_async_copy(
            w_hbm.at[pl.ds(_base(k), CHUNK)],
            bufs.at[k % NBUF],
            sin_ref.at[k % NBUF],
        )

    def issue_scatters(k):
        return [
            pltpu.make_async_copy(
                bufs.at[k % NBUF],
                o_hbm.at[pl.ds(b * N + _base(k), CHUNK)],
                sout[b].at[k % NBUF],
            )
            for b in range(BATCH)
        ]

    gathers = {}
    for k in range(LOOKAHEAD):
        gathers[k] = issue_gather(k)
        gathers[k].start()
    scatters = {}
    for g in range(NBLK):
        if g - LOOKAHEAD in scatters:
            for h in scatters.pop(g - LOOKAHEAD):
                h.wait()
        if g + LOOKAHEAD < NBLK:
            gathers[g + LOOKAHEAD] = issue_gather(g + LOOKAHEAD)
            gathers[g + LOOKAHEAD].start()
        gathers.pop(g).wait()

        slot = g % NBUF
        bufs[slot] = bufs[slot] * SCALE

        hs = issue_scatters(g)
        for h in hs:
            h.start()
        scatters[g] = hs

    for hs in scatters.values():
        for h in hs:
            h.wait()


def _tc_embed(w_flat):
    return pl.pallas_call(
        _body,
        in_specs=[pl.BlockSpec(memory_space=pltpu.HBM)],
        out_specs=pl.BlockSpec(memory_space=pltpu.HBM),
        out_shape=jax.ShapeDtypeStruct((BATCH * N,), jnp.float32),
        scratch_shapes=[
            pltpu.VMEM((NBUF, CHUNK), jnp.float32),
            pltpu.SemaphoreType.DMA((NBUF,)),
            pltpu.SemaphoreType.DMA((NBUF,)),
            pltpu.SemaphoreType.DMA((NBUF,)),
            pltpu.SemaphoreType.DMA((NBUF,)),
            pltpu.SemaphoreType.DMA((NBUF,)),
        ],
    )(w_flat)


def kernel(input, weights):
    del input
    out_flat = _tc_embed(weights.reshape(N))
    return out_flat.reshape(BATCH, ROWS, D)


# speedup vs baseline: 3.2835x; 3.2835x over previous
import math

import jax
import jax.numpy as jnp
from jax.experimental import pallas as pl
from jax.experimental.pallas import tpu as pltpu

D = 1024
ROWS = 2 * 4096 - 1
BATCH = 4
BLOCK_ROWS = 512
GRID = (ROWS + BLOCK_ROWS - 1) // BLOCK_ROWS
SCALE = math.sqrt(D)


def _body(w_ref, o0, o1, o2, o3):
    v = w_ref[...] * SCALE
    o0[...] = v
    o1[...] = v
    o2[...] = v
    o3[...] = v


def _tc_embed(weights):
    spec = pl.BlockSpec((BLOCK_ROWS, D), lambda i: (i, 0))
    shp = jax.ShapeDtypeStruct((ROWS, D), jnp.float32)
    outs = pl.pallas_call(
        _body,
        grid=(GRID,),
        in_specs=[spec],
        out_specs=[spec] * BATCH,
        out_shape=[shp] * BATCH,
        compiler_params=pltpu.CompilerParams(
            dimension_semantics=("arbitrary",),
        ),
    )(weights)
    return jnp.stack(outs, axis=0)


def kernel(input, weights):
    del input
    return _tc_embed(weights)


# speedup vs baseline: 4.1142x; 1.2530x over previous
import math

import jax
import jax.numpy as jnp
from jax.experimental import pallas as pl
from jax.experimental.pallas import tpu as pltpu

D = 1024
ROWS = 2 * 4096 - 1
BATCH = 4
BLOCK_ROWS = 512
GRID = (ROWS + BLOCK_ROWS - 1) // BLOCK_ROWS
SCALE = math.sqrt(D)


def _inner(w_blk, o0, o1, o2, o3):
    v = w_blk[...] * SCALE
    o0[...] = v
    o1[...] = v
    o2[...] = v
    o3[...] = v


def _body(w_hbm, o_hbm):
    spec = pl.BlockSpec((BLOCK_ROWS, D), lambda i: (i, 0))
    pltpu.emit_pipeline(
        _inner,
        grid=(GRID,),
        in_specs=[spec],
        out_specs=[spec] * BATCH,
    )(w_hbm, o_hbm.at[0], o_hbm.at[1], o_hbm.at[2], o_hbm.at[3])


def _tc_embed(weights):
    return pl.pallas_call(
        _body,
        in_specs=[pl.BlockSpec(memory_space=pltpu.HBM)],
        out_specs=pl.BlockSpec(memory_space=pltpu.HBM),
        out_shape=jax.ShapeDtypeStruct((BATCH, ROWS, D), jnp.float32),
    )(weights)


def kernel(input, weights):
    del input
    return _tc_embed(weights)
